# baseline (speedup 1.0000x reference)
import functools

import jax
import jax.numpy as jnp
from jax.experimental import pallas as pl
from jax.experimental.pallas import tpu as pltpu

B, QLEN, MLEN, DIM = 8, 4, 2048, 1024
HEADS, DIM_HEAD = 16, 64
INNER = HEADS * DIM_HEAD
TOPK, CHUNK = 8, 32
NCHUNK = MLEN // CHUNK
SCALE = DIM ** -0.5
HSCALE = DIM_HEAD ** -0.5
NEG = -1e30

_HI = jax.lax.Precision.HIGHEST


def _route_kernel(q_ref, mem_ref, wsq_ref, bsq_ref, wsk_ref, bsk_ref,
                  idx_ref, w_ref):
    mem = mem_ref[0]
    summ = mem.reshape(NCHUNK, CHUNK, DIM).mean(axis=1)
    sk = jax.lax.dot(summ, wsk_ref[...], precision=_HI) + bsk_ref[...]
    sq = jax.lax.dot(q_ref[0], wsq_ref[...], precision=_HI) + bsq_ref[...]
    sim = jax.lax.dot(sq, sk.T, precision=_HI) * SCALE

    col = jax.lax.broadcasted_iota(jnp.int32, (QLEN, NCHUNK), 1)
    work = sim
    logits, idxs = [], []
    for _ in range(TOPK):
        m = work.max(axis=1, keepdims=True)
        eq = work == m
        idx = jnp.min(jnp.where(eq, col, NCHUNK), axis=1, keepdims=True)
        logits.append(m)
        idxs.append(idx)
        work = jnp.where(col == idx, NEG, work)
    lg = jnp.concatenate(logits, axis=1)
    ii = jnp.concatenate(idxs, axis=1)
    e = jnp.exp(lg - lg.max(axis=1, keepdims=True))
    w = e / e.sum(axis=1, keepdims=True)
    idx_ref[0] = ii
    w_ref[0] = w


def _attend_kernel(idx_ref, c0, c1, c2, c3, c4, c5, c6, c7,
                   q_ref, w_all_ref, wq_ref, wkv_ref, wo_ref, bo_ref,
                   pos_ref, out_ref):
    b = pl.program_id(0)
    i = pl.program_id(1)

    chunks = jnp.concatenate(
        [c[0] for c in (c0, c1, c2, c3, c4, c5, c6, c7)], axis=0
    )
    kv = jax.lax.dot(chunks + pos_ref[...], wkv_ref[...],
                     precision=_HI)
    kk = kv[:, :INNER]
    vv = kv[:, INNER:]

    q = q_ref[0, pl.ds(i, 1), :]
    qp = jax.lax.dot(q, wq_ref[...], precision=_HI) * HSCALE

    lane = jax.lax.broadcasted_iota(jnp.int32, (INNER, HEADS), 0)
    hcol = jax.lax.broadcasted_iota(jnp.int32, (INNER, HEADS), 1)
    mask = (lane // DIM_HEAD == hcol).astype(jnp.float32)
    qmask = qp.reshape(INNER, 1) * mask

    scores = jax.lax.dot(kk, qmask, precision=_HI)
    s = scores.reshape(TOPK, CHUNK, HEADS)
    m = s.max(axis=1, keepdims=True)
    e = jnp.exp(s - m)
    p = e / e.sum(axis=1, keepdims=True)

    wsel = w_all_ref[pl.ds(b, 1), pl.ds(i, 1), :].reshape(TOPK, 1, 1)
    pw = (p * wsel).reshape(TOPK * CHUNK, HEADS)
    pexp = jax.lax.dot(pw, mask.T, precision=_HI)
    ovec = (pexp * vv).sum(axis=0, keepdims=True)

    out = jax.lax.dot(ovec, wo_ref[...], precision=_HI) + bo_ref[...]
    out_ref[0, pl.ds(i, 1), :] = out


def kernel(queries, memories, W_sq, b_sq, W_sk, b_sk, W_q, W_kv, W_o, b_o):
    b_sq2 = b_sq.reshape(1, DIM)
    b_sk2 = b_sk.reshape(1, DIM)
    b_o2 = b_o.reshape(1, DIM)

    idx, w = pl.pallas_call(
        _route_kernel,
        grid=(B,),
        in_specs=[
            pl.BlockSpec((1, QLEN, DIM), lambda b: (b, 0, 0)),
            pl.BlockSpec((1, MLEN, DIM), lambda b: (b, 0, 0)),
            pl.BlockSpec((DIM, DIM), lambda b: (0, 0)),
            pl.BlockSpec((1, DIM), lambda b: (0, 0)),
            pl.BlockSpec((DIM, DIM), lambda b: (0, 0)),
            pl.BlockSpec((1, DIM), lambda b: (0, 0)),
        ],
        out_specs=[
            pl.BlockSpec((1, QLEN, TOPK), lambda b: (b, 0, 0)),
            pl.BlockSpec((1, QLEN, TOPK), lambda b: (b, 0, 0)),
        ],
        out_shape=[
            jax.ShapeDtypeStruct((B, QLEN, TOPK), jnp.int32),
            jax.ShapeDtypeStruct((B, QLEN, TOPK), jnp.float32),
        ],
    )(queries, memories, W_sq, b_sq2, W_sk, b_sk2)

    idx_flat = idx.reshape(B * QLEN * TOPK)

    freqs = jnp.arange(0, DIM, 2.0)
    inv_freqs = 10000.0 ** (-freqs / DIM)
    seq = jnp.arange(CHUNK - 1, -1, -1.0)
    sinu = seq[:, None] * inv_freqs[None, :]
    pos = jnp.concatenate([jnp.sin(sinu), jnp.cos(sinu)], axis=-1)
    pos_tiled = jnp.tile(pos, (TOPK, 1)).astype(jnp.float32)

    def chunk_map(j):
        def f(b, i, idx_ref):
            return (b, idx_ref[(b * QLEN + i) * TOPK + j], 0)
        return f

    grid_spec = pltpu.PrefetchScalarGridSpec(
        num_scalar_prefetch=1,
        grid=(B, QLEN),
        in_specs=[
            *[pl.BlockSpec((1, CHUNK, DIM), chunk_map(j)) for j in range(TOPK)],
            pl.BlockSpec((1, QLEN, DIM), lambda b, i, s: (b, 0, 0)),
            pl.BlockSpec((B, QLEN, TOPK), lambda b, i, s: (0, 0, 0)),
            pl.BlockSpec((DIM, INNER), lambda b, i, s: (0, 0)),
            pl.BlockSpec((DIM, 2 * INNER), lambda b, i, s: (0, 0)),
            pl.BlockSpec((INNER, DIM), lambda b, i, s: (0, 0)),
            pl.BlockSpec((1, DIM), lambda b, i, s: (0, 0)),
            pl.BlockSpec((TOPK * CHUNK, DIM), lambda b, i, s: (0, 0)),
        ],
        out_specs=pl.BlockSpec((1, QLEN, DIM), lambda b, i, s: (b, 0, 0)),
    )

    out = pl.pallas_call(
        _attend_kernel,
        grid_spec=grid_spec,
        out_shape=jax.ShapeDtypeStruct((B, QLEN, DIM), jnp.float32),
    )(idx_flat,
      *([memories] * TOPK),
      queries, w, W_q, W_kv, W_o, b_o2, pos_tiled)

    return out


# speedup vs baseline: 3.2068x; 3.2068x over previous
import functools

import jax
import jax.numpy as jnp
from jax.experimental import pallas as pl
from jax.experimental.pallas import tpu as pltpu

B, QLEN, MLEN, DIM = 8, 4, 2048, 1024
HEADS, DIM_HEAD = 16, 64
INNER = HEADS * DIM_HEAD
TOPK, CHUNK = 8, 32
NCHUNK = MLEN // CHUNK
SCALE = DIM ** -0.5
HSCALE = DIM_HEAD ** -0.5
NEG = -1e30

_HI = jax.lax.Precision.HIGHEST


def _route_kernel(q_ref, mem_ref, wsq_ref, bsq_ref, wsk_ref, bsk_ref,
                  idx_ref, w_ref):
    mem = mem_ref[0]
    summ = mem.reshape(NCHUNK, CHUNK, DIM).mean(axis=1)
    sk = jax.lax.dot(summ, wsk_ref[...], precision=_HI) + bsk_ref[...]
    sq = jax.lax.dot(q_ref[0], wsq_ref[...], precision=_HI) + bsq_ref[...]
    sim = jax.lax.dot(sq, sk.T, precision=_HI) * SCALE

    col = jax.lax.broadcasted_iota(jnp.int32, (QLEN, NCHUNK), 1)
    work = sim
    logits, idxs = [], []
    for _ in range(TOPK):
        m = work.max(axis=1, keepdims=True)
        eq = work == m
        idx = jnp.min(jnp.where(eq, col, NCHUNK), axis=1, keepdims=True)
        logits.append(m)
        idxs.append(idx)
        work = jnp.where(col == idx, NEG, work)
    lg = jnp.concatenate(logits, axis=1)
    ii = jnp.concatenate(idxs, axis=1)
    e = jnp.exp(lg - lg.max(axis=1, keepdims=True))
    w = e / e.sum(axis=1, keepdims=True)
    idx_ref[0] = ii
    w_ref[0] = w


def _attend_kernel(idx_ref, c0, c1, c2, c3, c4, c5, c6, c7,
                   q_ref, w_all_ref, wq_ref, wkv_ref, wo_ref, bo_ref,
                   pos_ref, out_ref):
    b = pl.program_id(0)
    i = pl.program_id(1)

    chunks = jnp.concatenate(
        [c[0] for c in (c0, c1, c2, c3, c4, c5, c6, c7)], axis=0
    )
    kv = jax.lax.dot(chunks + pos_ref[...], wkv_ref[...])
    kk = kv[:, :INNER]
    vv = kv[:, INNER:]

    q = q_ref[0, pl.ds(i, 1), :]
    qp = jax.lax.dot(q, wq_ref[...]) * HSCALE

    lane = jax.lax.broadcasted_iota(jnp.int32, (INNER, HEADS), 0)
    hcol = jax.lax.broadcasted_iota(jnp.int32, (INNER, HEADS), 1)
    mask = (lane // DIM_HEAD == hcol).astype(jnp.float32)
    qmask = qp.reshape(INNER, 1) * mask

    scores = jax.lax.dot(kk, qmask)
    s = scores.reshape(TOPK, CHUNK, HEADS)
    m = s.max(axis=1, keepdims=True)
    e = jnp.exp(s - m)
    p = e / e.sum(axis=1, keepdims=True)

    wsel = w_all_ref[pl.ds(b, 1), pl.ds(i, 1), :].reshape(TOPK, 1, 1)
    pw = (p * wsel).reshape(TOPK * CHUNK, HEADS)
    pexp = jax.lax.dot(pw, mask.T)
    ovec = (pexp * vv).sum(axis=0, keepdims=True)

    out = jax.lax.dot(ovec, wo_ref[...]) + bo_ref[...]
    out_ref[0, pl.ds(i, 1), :] = out


def kernel(queries, memories, W_sq, b_sq, W_sk, b_sk, W_q, W_kv, W_o, b_o):
    b_sq2 = b_sq.reshape(1, DIM)
    b_sk2 = b_sk.reshape(1, DIM)
    b_o2 = b_o.reshape(1, DIM)

    idx, w = pl.pallas_call(
        _route_kernel,
        grid=(B,),
        in_specs=[
            pl.BlockSpec((1, QLEN, DIM), lambda b: (b, 0, 0)),
            pl.BlockSpec((1, MLEN, DIM), lambda b: (b, 0, 0)),
            pl.BlockSpec((DIM, DIM), lambda b: (0, 0)),
            pl.BlockSpec((1, DIM), lambda b: (0, 0)),
            pl.BlockSpec((DIM, DIM), lambda b: (0, 0)),
            pl.BlockSpec((1, DIM), lambda b: (0, 0)),
        ],
        out_specs=[
            pl.BlockSpec((1, QLEN, TOPK), lambda b: (b, 0, 0)),
            pl.BlockSpec((1, QLEN, TOPK), lambda b: (b, 0, 0)),
        ],
        out_shape=[
            jax.ShapeDtypeStruct((B, QLEN, TOPK), jnp.int32),
            jax.ShapeDtypeStruct((B, QLEN, TOPK), jnp.float32),
        ],
    )(queries, memories, W_sq, b_sq2, W_sk, b_sk2)

    idx_flat = idx.reshape(B * QLEN * TOPK)

    freqs = jnp.arange(0, DIM, 2.0)
    inv_freqs = 10000.0 ** (-freqs / DIM)
    seq = jnp.arange(CHUNK - 1, -1, -1.0)
    sinu = seq[:, None] * inv_freqs[None, :]
    pos = jnp.concatenate([jnp.sin(sinu), jnp.cos(sinu)], axis=-1)
    pos_tiled = jnp.tile(pos, (TOPK, 1)).astype(jnp.float32)

    def chunk_map(j):
        def f(b, i, idx_ref):
            return (b, idx_ref[(b * QLEN + i) * TOPK + j], 0)
        return f

    grid_spec = pltpu.PrefetchScalarGridSpec(
        num_scalar_prefetch=1,
        grid=(B, QLEN),
        in_specs=[
            *[pl.BlockSpec((1, CHUNK, DIM), chunk_map(j)) for j in range(TOPK)],
            pl.BlockSpec((1, QLEN, DIM), lambda b, i, s: (b, 0, 0)),
            pl.BlockSpec((B, QLEN, TOPK), lambda b, i, s: (0, 0, 0)),
            pl.BlockSpec((DIM, INNER), lambda b, i, s: (0, 0)),
            pl.BlockSpec((DIM, 2 * INNER), lambda b, i, s: (0, 0)),
            pl.BlockSpec((INNER, DIM), lambda b, i, s: (0, 0)),
            pl.BlockSpec((1, DIM), lambda b, i, s: (0, 0)),
            pl.BlockSpec((TOPK * CHUNK, DIM), lambda b, i, s: (0, 0)),
        ],
        out_specs=pl.BlockSpec((1, QLEN, DIM), lambda b, i, s: (b, 0, 0)),
    )

    out = pl.pallas_call(
        _attend_kernel,
        grid_spec=grid_spec,
        out_shape=jax.ShapeDtypeStruct((B, QLEN, DIM), jnp.float32),
    )(idx_flat,
      *([memories] * TOPK),
      queries, w, W_q, W_kv, W_o, b_o2, pos_tiled)

    return out
